# baseline (speedup 1.0000x reference)
import functools

import jax
import jax.numpy as jnp
from jax import lax
from jax.experimental import pallas as pl
from jax.experimental.pallas import tpu as pltpu


def _pick_tile(b, target=256):
    best = 8
    for t in range(8, min(target, b) + 1, 8):
        if b % t == 0:
            best = t
    return best


def _outer_body(x_hbm, x2_hbm, wh_ref, bh_ref, wp_ref, bp_ref,
                o1a, o2a, o1b, o2b, *, feat_dim, tb, steps):
    def _head(xv):
        f = jnp.dot(xv, wh_ref[...], preferred_element_type=jnp.float32)
        return jnp.maximum(f + bh_ref[...], 0.0)

    def _proj_norm(f, o1_ref, o2_ref):
        y = jnp.dot(f, wp_ref[...], preferred_element_type=jnp.float32) + bp_ref[...]
        y1 = y[:, :feat_dim]
        y2 = y[:, feat_dim:]
        o1_ref[...] = (y1 * lax.rsqrt(jnp.sum(y1 * y1, axis=-1, keepdims=True))
                       ).astype(o1_ref.dtype)
        o2_ref[...] = (y2 * lax.rsqrt(jnp.sum(y2 * y2, axis=-1, keepdims=True))
                       ).astype(o2_ref.dtype)

    def _step(x_ref, x2_ref, o1a_ref, o2a_ref, o1b_ref, o2b_ref):
        fa = _head(x_ref[...])
        fb = _head(x2_ref[...])
        _proj_norm(fa, o1a_ref, o2a_ref)
        _proj_norm(fb, o1b_ref, o2b_ref)

    D = wh_ref.shape[0]
    pipe = pltpu.emit_pipeline(
        _step,
        grid=(steps,),
        in_specs=[
            pl.BlockSpec((tb, D), lambda i: (i, 0)),
            pl.BlockSpec((tb, D), lambda i: (i, 0)),
        ],
        out_specs=[
            pl.BlockSpec((tb, feat_dim), lambda i: (i, 0)),
            pl.BlockSpec((tb, feat_dim), lambda i: (i, 0)),
            pl.BlockSpec((tb, feat_dim), lambda i: (i, 0)),
            pl.BlockSpec((tb, feat_dim), lambda i: (i, 0)),
        ],
    )
    pipe(x_hbm, x2_hbm, o1a, o2a, o1b, o2b)


@jax.jit
def _run(x, x2, w_head, b_head, w_proj, b_proj):
    B, D = x.shape
    F2 = w_proj.shape[1]
    feat_dim = F2 // 2
    tb = _pick_tile(B)
    steps = B // tb
    any_spec = pl.BlockSpec(memory_space=pltpu.MemorySpace.HBM)
    vmem_spec = pl.BlockSpec(memory_space=pltpu.MemorySpace.VMEM)
    return pl.pallas_call(
        functools.partial(_outer_body, feat_dim=feat_dim, tb=tb, steps=steps),
        out_shape=tuple(jax.ShapeDtypeStruct((B, feat_dim), x.dtype)
                        for _ in range(4)),
        in_specs=[any_spec, any_spec, vmem_spec, vmem_spec, vmem_spec, vmem_spec],
        out_specs=(any_spec, any_spec, any_spec, any_spec),
        compiler_params=pltpu.CompilerParams(
            vmem_limit_bytes=100 * 1024 * 1024,
        ),
    )(x, x2, w_head, b_head, w_proj, b_proj)


def kernel(x, x2, w_head, b_head, w_proj, b_proj):
    return _run(x, x2, w_head, b_head, w_proj, b_proj)


# speedup vs baseline: 1.0332x; 1.0332x over previous
import functools

import jax
import jax.numpy as jnp
from jax import lax
from jax.experimental import pallas as pl
from jax.experimental.pallas import tpu as pltpu


def _pick_tile(b, target=512):
    best = 8
    for t in range(8, min(target, b) + 1, 8):
        if b % t == 0:
            best = t
    return best


def _outer_body(x_hbm, x2_hbm, wh_ref, bh_ref, wp_ref, bp_ref,
                o1a, o2a, o1b, o2b, *, feat_dim, tb, steps):
    def _head(xv):
        f = jnp.dot(xv, wh_ref[...], preferred_element_type=jnp.float32)
        return jnp.maximum(f + bh_ref[...], 0.0)

    def _proj_norm(f, o1_ref, o2_ref):
        y = jnp.dot(f, wp_ref[...], preferred_element_type=jnp.float32) + bp_ref[...]
        y1 = y[:, :feat_dim]
        y2 = y[:, feat_dim:]
        o1_ref[...] = (y1 * lax.rsqrt(jnp.sum(y1 * y1, axis=-1, keepdims=True))
                       ).astype(o1_ref.dtype)
        o2_ref[...] = (y2 * lax.rsqrt(jnp.sum(y2 * y2, axis=-1, keepdims=True))
                       ).astype(o2_ref.dtype)

    def _step(x_ref, x2_ref, o1a_ref, o2a_ref, o1b_ref, o2b_ref):
        fa = _head(x_ref[...])
        fb = _head(x2_ref[...])
        _proj_norm(fa, o1a_ref, o2a_ref)
        _proj_norm(fb, o1b_ref, o2b_ref)

    D = wh_ref.shape[0]
    pipe = pltpu.emit_pipeline(
        _step,
        grid=(steps,),
        in_specs=[
            pl.BlockSpec((tb, D), lambda i: (i, 0)),
            pl.BlockSpec((tb, D), lambda i: (i, 0)),
        ],
        out_specs=[
            pl.BlockSpec((tb, feat_dim), lambda i: (i, 0)),
            pl.BlockSpec((tb, feat_dim), lambda i: (i, 0)),
            pl.BlockSpec((tb, feat_dim), lambda i: (i, 0)),
            pl.BlockSpec((tb, feat_dim), lambda i: (i, 0)),
        ],
    )
    pipe(x_hbm, x2_hbm, o1a, o2a, o1b, o2b)


@jax.jit
def _run(x, x2, w_head, b_head, w_proj, b_proj):
    B, D = x.shape
    F2 = w_proj.shape[1]
    feat_dim = F2 // 2
    tb = _pick_tile(B)
    steps = B // tb
    any_spec = pl.BlockSpec(memory_space=pltpu.MemorySpace.HBM)
    vmem_spec = pl.BlockSpec(memory_space=pltpu.MemorySpace.VMEM)
    return pl.pallas_call(
        functools.partial(_outer_body, feat_dim=feat_dim, tb=tb, steps=steps),
        out_shape=tuple(jax.ShapeDtypeStruct((B, feat_dim), x.dtype)
                        for _ in range(4)),
        in_specs=[any_spec, any_spec, vmem_spec, vmem_spec, vmem_spec, vmem_spec],
        out_specs=(any_spec, any_spec, any_spec, any_spec),
        compiler_params=pltpu.CompilerParams(
            vmem_limit_bytes=100 * 1024 * 1024,
        ),
    )(x, x2, w_head, b_head, w_proj, b_proj)


def kernel(x, x2, w_head, b_head, w_proj, b_proj):
    return _run(x, x2, w_head, b_head, w_proj, b_proj)
